# baseline (speedup 1.0000x reference)
import jax
import jax.numpy as jnp
from jax.experimental import pallas as pl
from jax.experimental.pallas import tpu as pltpu


def _round_up(x: int, m: int) -> int:
    return ((x + m - 1) // m) * m


def _gate_kernel(xn_ref, xv_ref, xs_ref, w1_ref, b1_ref, w2_ref, b2_ref,
                 o_ref):
    x = jnp.concatenate(
        [xn_ref[...].astype(jnp.bfloat16),
         xv_ref[...].astype(jnp.bfloat16),
         xs_ref[...].astype(jnp.bfloat16)], axis=-1)

    cat = jnp.dot(x, w1_ref[...], preferred_element_type=jnp.float32)
    cat = cat + b1_ref[...]

    gated = (0.5 * jnp.tanh(0.5 * cat) + 0.5) * cat

    fuse = jnp.dot(gated.astype(jnp.bfloat16), w2_ref[...],
                   preferred_element_type=jnp.float32)
    o_ref[...] = (fuse + b2_ref[...]).astype(o_ref.dtype)


def kernel(node_feature, vis_memory, sem_memory, w1, b1, w2, b2,
           *, tile_n=2048):
    N, node_size = node_feature.shape
    vis_size = vis_memory.shape[1]
    sem_size = sem_memory.shape[1]
    D = node_size + vis_size + sem_size
    O = w2.shape[1]
    out_dtype = node_feature.dtype

    sublane = {4: 8, 2: 16, 1: 32}[jnp.dtype(out_dtype).itemsize]

    half = _round_up(max(1, -(-N // 2)), sublane)
    tile_n = max(sublane, min(_round_up(int(tile_n), sublane), half))
    grid = (pl.cdiv(N, tile_n),)

    Dh = _round_up(D, 128)
    w1p = jnp.pad(w1, ((0, 0), (0, Dh - D))).astype(jnp.bfloat16)
    b1p = jnp.pad(b1, (0, Dh - D)).reshape(1, Dh).astype(jnp.float32)
    w2p = jnp.pad(w2, ((0, Dh - D), (0, 0))).astype(jnp.bfloat16)
    b2p = b2.reshape(1, O).astype(jnp.float32)

    out = pl.pallas_call(
        _gate_kernel,
        out_shape=jax.ShapeDtypeStruct((N, O), out_dtype),
        grid=grid,
        in_specs=[
            pl.BlockSpec((tile_n, node_size), lambda i: (i, 0)),
            pl.BlockSpec((tile_n, vis_size), lambda i: (i, 0)),
            pl.BlockSpec((tile_n, sem_size), lambda i: (i, 0)),
            pl.BlockSpec((D, Dh), lambda i: (0, 0)),
            pl.BlockSpec((1, Dh), lambda i: (0, 0)),
            pl.BlockSpec((Dh, O), lambda i: (0, 0)),
            pl.BlockSpec((1, O), lambda i: (0, 0)),
        ],
        out_specs=pl.BlockSpec((tile_n, O), lambda i: (i, 0)),
        compiler_params=pltpu.CompilerParams(
            dimension_semantics=("parallel",)),
    )(node_feature, vis_memory, sem_memory, w1p, b1p, w2p, b2p)

    return out


# speedup vs baseline: 1.0278x; 1.0278x over previous
import jax
import jax.numpy as jnp
from jax.experimental import pallas as pl
from jax.experimental.pallas import tpu as pltpu


def _round_up(x: int, m: int) -> int:
    return ((x + m - 1) // m) * m


def _gate_kernel(ns, vs, xn_ref, xv_ref, xs_ref, w1_ref, b1_ref,
                 w2_ref, b2_ref, o_ref):
    ss = w1_ref.shape[0] - ns - vs
    cat = (jnp.dot(xn_ref[...], w1_ref[pl.ds(0, ns), :],
                   preferred_element_type=jnp.float32)
           + jnp.dot(xv_ref[...], w1_ref[pl.ds(ns, vs), :],
                     preferred_element_type=jnp.float32)
           + jnp.dot(xs_ref[...], w1_ref[pl.ds(ns + vs, ss), :],
                     preferred_element_type=jnp.float32)
           + b1_ref[...])

    gated = (0.5 * jnp.tanh(0.5 * cat) + 0.5) * cat

    fuse = jnp.dot(gated, w2_ref[...], preferred_element_type=jnp.float32)
    o_ref[...] = (fuse + b2_ref[...]).astype(o_ref.dtype)


def kernel(node_feature, vis_memory, sem_memory, w1, b1, w2, b2,
           *, tile_n=2048):
    N, node_size = node_feature.shape
    vis_size = vis_memory.shape[1]
    sem_size = sem_memory.shape[1]
    D = node_size + vis_size + sem_size
    O = w2.shape[1]
    out_dtype = node_feature.dtype

    sublane = {4: 8, 2: 16, 1: 32}[jnp.dtype(out_dtype).itemsize]
    tile_n = max(sublane, min(_round_up(int(tile_n), sublane), N))
    grid = (pl.cdiv(N, tile_n),)

    Dh = _round_up(D, 128)
    if Dh != D:
        w1p = jnp.pad(w1, ((0, 0), (0, Dh - D)))
        b1v = jnp.pad(b1, (0, Dh - D))
        w2p = jnp.pad(w2, ((0, Dh - D), (0, 0)))
    else:
        w1p, b1v, w2p = w1, b1, w2
    b1p = b1v.reshape(1, Dh)
    b2p = b2.reshape(1, O)

    import functools
    body = functools.partial(_gate_kernel, node_size, vis_size)

    out = pl.pallas_call(
        body,
        out_shape=jax.ShapeDtypeStruct((N, O), out_dtype),
        grid=grid,
        in_specs=[
            pl.BlockSpec((tile_n, node_size), lambda i: (i, 0)),
            pl.BlockSpec((tile_n, vis_size), lambda i: (i, 0)),
            pl.BlockSpec((tile_n, sem_size), lambda i: (i, 0)),
            pl.BlockSpec((D, Dh), lambda i: (0, 0)),
            pl.BlockSpec((1, Dh), lambda i: (0, 0)),
            pl.BlockSpec((Dh, O), lambda i: (0, 0)),
            pl.BlockSpec((1, O), lambda i: (0, 0)),
        ],
        out_specs=pl.BlockSpec((tile_n, O), lambda i: (i, 0)),
        compiler_params=pltpu.CompilerParams(
            dimension_semantics=("arbitrary",)),
    )(node_feature, vis_memory, sem_memory, w1p, b1p, w2p, b2p)

    return out


# speedup vs baseline: 1.1151x; 1.0850x over previous
import functools

import jax
import jax.numpy as jnp
from jax.experimental import pallas as pl
from jax.experimental.pallas import tpu as pltpu


def _round_up(x: int, m: int) -> int:
    return ((x + m - 1) // m) * m


_MRB_M = 1024


def _gate_kernel_mxu(ns, vs, ss, xn_ref, xv_ref, xs_ref, w1_ref, b1_ref,
                     w2_ref, b2_ref, o_ref):
    D = ns + vs + ss
    O = o_ref.shape[1]
    rows = o_ref.shape[0]

    ktiles = []
    for ref, width in ((xn_ref, ns), (xv_ref, vs), (xs_ref, ss)):
        for c in range(0, width, 256):
            ktiles.append((ref, c, len(ktiles) * 256))

    n1 = D // 256
    n2 = O // 256

    for m0 in range(0, rows, _MRB_M):
        M = min(_MRB_M, rows - m0)
        r = pl.ds(m0, M)

        gated = []
        for nt in range(n1):
            mxu = nt % 2
            for j, (ref, c, woff) in enumerate(ktiles):
                w_tile = w1_ref[pl.ds(woff, 256), pl.ds(nt * 256, 256)]
                pltpu.matmul_push_rhs(w_tile, staging_register=j % 2,
                                      mxu_index=mxu)
                pltpu.matmul_acc_lhs(0, ref[r, pl.ds(c, 256)],
                                     mxu_index=mxu, load_staged_rhs=j % 2)
            cat = pltpu.matmul_pop(0, (M, 256), jnp.float32, mxu_index=mxu)
            cat = cat + b1_ref[:, pl.ds(nt * 256, 256)]
            gated.append((0.5 * jnp.tanh(0.5 * cat) + 0.5) * cat)

        for ot in range(n2):
            mxu = ot % 2
            for nt in range(n1):
                w_tile = w2_ref[pl.ds(nt * 256, 256), pl.ds(ot * 256, 256)]
                pltpu.matmul_push_rhs(w_tile, staging_register=nt % 2,
                                      mxu_index=mxu)
                pltpu.matmul_acc_lhs(0, gated[nt], mxu_index=mxu,
                                     load_staged_rhs=nt % 2)
            fuse = pltpu.matmul_pop(0, (M, 256), jnp.float32, mxu_index=mxu)
            o_ref[r, pl.ds(ot * 256, 256)] = (
                fuse + b2_ref[:, pl.ds(ot * 256, 256)]).astype(o_ref.dtype)


def _gate_kernel_dot(ns, vs, xn_ref, xv_ref, xs_ref, w1_ref, b1_ref,
                     w2_ref, b2_ref, o_ref):
    ss = w1_ref.shape[0] - ns - vs
    cat = (jnp.dot(xn_ref[...], w1_ref[pl.ds(0, ns), :],
                   preferred_element_type=jnp.float32)
           + jnp.dot(xv_ref[...], w1_ref[pl.ds(ns, vs), :],
                     preferred_element_type=jnp.float32)
           + jnp.dot(xs_ref[...], w1_ref[pl.ds(ns + vs, ss), :],
                     preferred_element_type=jnp.float32)
           + b1_ref[...])
    gated = (0.5 * jnp.tanh(0.5 * cat) + 0.5) * cat
    fuse = jnp.dot(gated, w2_ref[...], preferred_element_type=jnp.float32)
    o_ref[...] = (fuse + b2_ref[...]).astype(o_ref.dtype)


def kernel(node_feature, vis_memory, sem_memory, w1, b1, w2, b2,
           *, tile_n=2048):
    N, node_size = node_feature.shape
    vis_size = vis_memory.shape[1]
    sem_size = sem_memory.shape[1]
    D = node_size + vis_size + sem_size
    O = w2.shape[1]
    out_dtype = node_feature.dtype

    sublane = {4: 8, 2: 16, 1: 32}[jnp.dtype(out_dtype).itemsize]
    tile_n = max(sublane, min(_round_up(int(tile_n), sublane), N))

    use_mxu = (node_size % 256 == 0 and vis_size % 256 == 0
               and sem_size % 256 == 0 and D % 256 == 0 and O % 256 == 0
               and tile_n % _MRB_M == 0 and N % tile_n == 0
               and out_dtype == jnp.float32)

    grid = (pl.cdiv(N, tile_n),)

    Dh = _round_up(D, 128)
    if Dh != D:
        w1p = jnp.pad(w1, ((0, 0), (0, Dh - D)))
        b1v = jnp.pad(b1, (0, Dh - D))
        w2p = jnp.pad(w2, ((0, Dh - D), (0, 0)))
        use_mxu = False
    else:
        w1p, b1v, w2p = w1, b1, w2
    b1p = b1v.reshape(1, Dh)
    b2p = b2.reshape(1, O)

    if use_mxu:
        body = functools.partial(_gate_kernel_mxu, node_size, vis_size,
                                 sem_size)
    else:
        body = functools.partial(_gate_kernel_dot, node_size, vis_size)

    out = pl.pallas_call(
        body,
        out_shape=jax.ShapeDtypeStruct((N, O), out_dtype),
        grid=grid,
        in_specs=[
            pl.BlockSpec((tile_n, node_size), lambda i: (i, 0)),
            pl.BlockSpec((tile_n, vis_size), lambda i: (i, 0)),
            pl.BlockSpec((tile_n, sem_size), lambda i: (i, 0)),
            pl.BlockSpec((D, Dh), lambda i: (0, 0)),
            pl.BlockSpec((1, Dh), lambda i: (0, 0)),
            pl.BlockSpec((Dh, O), lambda i: (0, 0)),
            pl.BlockSpec((1, O), lambda i: (0, 0)),
        ],
        out_specs=pl.BlockSpec((tile_n, O), lambda i: (i, 0)),
        compiler_params=pltpu.CompilerParams(
            dimension_semantics=("arbitrary",)),
    )(node_feature, vis_memory, sem_memory, w1p, b1p, w2p, b2p)

    return out
